# baseline (speedup 1.0000x reference)
import functools

import jax
import jax.numpy as jnp
from jax import lax
from jax.experimental import pallas as pl
from jax.experimental.pallas import tpu as pltpu
from jax.experimental.pallas import tpu_sc as plsc

B = 32
N = 2048
NP = 128
PP = 32
ROWS = B * NP * PP
BLK = 1024
GRID = ROWS // BLK


def _trunc_bf16(v):
    bits = plsc.bitcast(v, jnp.int32)
    rounded = bits + 0x7FFF + ((bits >> 16) & 1)
    return plsc.bitcast(rounded & jnp.int32(-65536), jnp.float32)


def _sc_body(xs_h, ys_h, zs_h,
             cx_h, cy_h, cz_h, px_h, py_h, pz_h,
             xs_v, ys_v, zs_v, dist_v, xb_v, yb_v, zb_v, sp_v,
             cx_v, cy_v, cz_v, px_v, py_v, pz_v):
    w = lax.axis_index("s") * 2 + lax.axis_index("c")
    pltpu.sync_copy(xs_h.at[w], xs_v)
    pltpu.sync_copy(ys_h.at[w], ys_v)
    pltpu.sync_copy(zs_h.at[w], zs_v)

    iota = lax.iota(jnp.int32, 16)
    lane0 = iota == 0
    inf16 = jnp.full((16,), 1e30, jnp.float32)
    zero16i = jnp.zeros((16,), jnp.int32)

    def init_chunk(c, carry):
        sl = pl.ds(c * 16, 16)
        dist_v[sl] = jnp.full((16,), 1e10, jnp.float32)
        xv = xs_v[sl]
        yv = ys_v[sl]
        zv = zs_v[sl]
        xb_v[sl] = _trunc_bf16(xv)
        yb_v[sl] = _trunc_bf16(yv)
        zb_v[sl] = _trunc_bf16(zv)
        sp_v[sl] = xv * xv + yv * yv + zv * zv
        return carry

    lax.fori_loop(0, N // 16, init_chunk, 0)

    def fps_step(t, far):
        fv = jnp.full((16,), far, jnp.int32)
        cxv = plsc.load_gather(xs_v, [fv])
        cyv = plsc.load_gather(ys_v, [fv])
        czv = plsc.load_gather(zs_v, [fv])
        tv = jnp.full((16,), t, jnp.int32)
        plsc.store_scatter(cx_v, [tv], cxv, mask=lane0)
        plsc.store_scatter(cy_v, [tv], cyv, mask=lane0)
        plsc.store_scatter(cz_v, [tv], czv, mask=lane0)

        def chunk(c, carry):
            m, mi = carry
            sl = pl.ds(c * 16, 16)
            dx = xs_v[sl] - cxv
            dy = ys_v[sl] - cyv
            dz = zs_v[sl] - czv
            d = dx * dx + dy * dy + dz * dz
            dn = jnp.minimum(dist_v[sl], d)
            dist_v[sl] = dn
            upd = dn > m
            m = jnp.where(upd, dn, m)
            mi = jnp.where(upd, c * 16 + iota, mi)
            return (m, mi)

        m, mi = lax.fori_loop(0, N // 16, chunk,
                              (jnp.full((16,), -1.0, jnp.float32), zero16i),
                              unroll=4)
        mmax = jnp.max(m)
        cand = jnp.where(m == mmax, mi, N)
        return jnp.min(cand)

    lax.fori_loop(0, NP, fps_step, jnp.int32(0))

    def knn_q(q, carry):
        qv = jnp.full((16,), q, jnp.int32)
        qx = plsc.load_gather(cx_v, [qv])
        qy = plsc.load_gather(cy_v, [qv])
        qz = plsc.load_gather(cz_v, [qv])
        qxb = _trunc_bf16(qx)
        qyb = _trunc_bf16(qy)
        qzb = _trunc_bf16(qz)
        sq = qx * qx + qy * qy + qz * qz

        def chunk(c, st):
            r0k, r0v, r1k, r1v = st
            sl = pl.ds(c * 16, 16)
            prod = xb_v[sl] * qxb + yb_v[sl] * qyb
            prod = prod + zb_v[sl] * qzb
            d = -2.0 * prod + sq
            d = d + sp_v[sl]
            ck, cv = plsc.sort_key_val(d, c * 16 + iota)
            rck = lax.rev(ck, (0,))
            rcv = lax.rev(cv, (0,))
            sel = r1k <= rck
            lk = jnp.where(sel, r1k, rck)
            lv = jnp.where(sel, r1v, rcv)
            lk, lv = plsc.sort_key_val(lk, lv)
            rlk = lax.rev(lk, (0,))
            rlv = lax.rev(lv, (0,))
            sel2 = r0k <= rlk
            lok = jnp.where(sel2, r0k, rlk)
            lov = jnp.where(sel2, r0v, rlv)
            hik = jnp.where(sel2, rlk, r0k)
            hiv = jnp.where(sel2, rlv, r0v)
            r0k, r0v = plsc.sort_key_val(lok, lov)
            r1k, r1v = plsc.sort_key_val(hik, hiv)
            return (r0k, r0v, r1k, r1v)

        r0k, r0v, r1k, r1v = lax.fori_loop(
            0, N // 16, chunk, (inf16, zero16i, inf16, zero16i), unroll=2)

        base = q * PP
        px_v[pl.ds(base, 16)] = plsc.load_gather(xs_v, [r0v]) - qx
        px_v[pl.ds(base + 16, 16)] = plsc.load_gather(xs_v, [r1v]) - qx
        py_v[pl.ds(base, 16)] = plsc.load_gather(ys_v, [r0v]) - qy
        py_v[pl.ds(base + 16, 16)] = plsc.load_gather(ys_v, [r1v]) - qy
        pz_v[pl.ds(base, 16)] = plsc.load_gather(zs_v, [r0v]) - qz
        pz_v[pl.ds(base + 16, 16)] = plsc.load_gather(zs_v, [r1v]) - qz
        return carry

    lax.fori_loop(0, NP, knn_q, 0)

    pltpu.sync_copy(cx_v, cx_h.at[w])
    pltpu.sync_copy(cy_v, cy_h.at[w])
    pltpu.sync_copy(cz_v, cz_h.at[w])
    pltpu.sync_copy(px_v, px_h.at[w])
    pltpu.sync_copy(py_v, py_h.at[w])
    pltpu.sync_copy(pz_v, pz_h.at[w])


_sc_call = functools.partial(
    pl.kernel,
    mesh=plsc.VectorSubcoreMesh(core_axis_name="c", subcore_axis_name="s"),
    compiler_params=pltpu.CompilerParams(needs_layout_passes=False),
    out_type=[
        jax.ShapeDtypeStruct((B, NP), jnp.float32),
        jax.ShapeDtypeStruct((B, NP), jnp.float32),
        jax.ShapeDtypeStruct((B, NP), jnp.float32),
        jax.ShapeDtypeStruct((B, NP * PP), jnp.float32),
        jax.ShapeDtypeStruct((B, NP * PP), jnp.float32),
        jax.ShapeDtypeStruct((B, NP * PP), jnp.float32),
    ],
    scratch_types=[
        pltpu.VMEM((N,), jnp.float32),
        pltpu.VMEM((N,), jnp.float32),
        pltpu.VMEM((N,), jnp.float32),
        pltpu.VMEM((N,), jnp.float32),
        pltpu.VMEM((N,), jnp.float32),
        pltpu.VMEM((N,), jnp.float32),
        pltpu.VMEM((N,), jnp.float32),
        pltpu.VMEM((N,), jnp.float32),
        pltpu.VMEM((NP,), jnp.float32),
        pltpu.VMEM((NP,), jnp.float32),
        pltpu.VMEM((NP,), jnp.float32),
        pltpu.VMEM((NP * PP,), jnp.float32),
        pltpu.VMEM((NP * PP,), jnp.float32),
        pltpu.VMEM((NP * PP,), jnp.float32),
    ],
)(_sc_body)


def _dotf(a, b):
    return lax.dot_general(a, b, (((1,), (0,)), ((), ())),
                           preferred_element_type=jnp.float32)


def _mom_body(p_ref, m_ref):
    @pl.when(pl.program_id(0) == 0)
    def _():
        m_ref[...] = jnp.zeros_like(m_ref)

    p = p_ref[...]
    m_ref[...] += lax.dot_general(p, p, (((0,), (0,)), ((), ())),
                                  preferred_element_type=jnp.float32)


def _a1_of(p_ref, w1_ref, m1_ref, sc1_ref, be1_ref):
    h1 = _dotf(p_ref[...], w1_ref[...])
    return jnp.maximum(
        (h1 - m1_ref[0:1, :]) * sc1_ref[0:1, :] + be1_ref[0:1, :], 0.0)


def _stat1_body(p_ref, w1_ref, m1_ref, sc1_ref, be1_ref, g_ref, s_ref):
    @pl.when(pl.program_id(0) == 0)
    def _():
        g_ref[...] = jnp.zeros_like(g_ref)
        s_ref[...] = jnp.zeros_like(s_ref)

    a1 = _a1_of(p_ref, w1_ref, m1_ref, sc1_ref, be1_ref).astype(jnp.bfloat16)
    g_ref[...] += lax.dot_general(a1, a1, (((0,), (0,)), ((), ())),
                                  preferred_element_type=jnp.float32)
    s_ref[...] += jnp.broadcast_to(
        jnp.sum(a1.astype(jnp.float32), axis=0, keepdims=True), s_ref.shape)


def _mlp_body(p_ref, w1_ref, m1_ref, sc1_ref, be1_ref,
              w2_ref, cb2_ref, sc2_ref, be2_ref, w3_ref, b3_ref, o_ref):
    a1 = _a1_of(p_ref, w1_ref, m1_ref, sc1_ref, be1_ref).astype(jnp.bfloat16)
    h2 = _dotf(a1, w2_ref[...])
    a2 = jnp.maximum(
        (h2 + cb2_ref[0:1, :]) * sc2_ref[0:1, :] + be2_ref[0:1, :], 0.0)
    h3 = _dotf(a2.astype(jnp.bfloat16), w3_ref[...]) + b3_ref[0:1, :]
    o_ref[...] = jnp.max(h3.reshape(BLK // PP, PP, h3.shape[-1]), axis=1)


def _row8(v, n):
    return jnp.broadcast_to(v[None, :], (8, n)).astype(jnp.float32)


def _moments(p_aug):
    return pl.pallas_call(
        _mom_body,
        grid=(GRID,),
        in_specs=[pl.BlockSpec((BLK, 8), lambda i: (i, 0))],
        out_specs=pl.BlockSpec((8, 8), lambda i: (0, 0)),
        out_shape=jax.ShapeDtypeStruct((8, 8), jnp.float32),
    )(p_aug)


def _stats1(p_aug, w1b, m1r, sc1r, be1r):
    return pl.pallas_call(
        _stat1_body,
        grid=(GRID,),
        in_specs=[
            pl.BlockSpec((BLK, 8), lambda i: (i, 0)),
            pl.BlockSpec((8, 64), lambda i: (0, 0)),
            pl.BlockSpec((8, 64), lambda i: (0, 0)),
            pl.BlockSpec((8, 64), lambda i: (0, 0)),
            pl.BlockSpec((8, 64), lambda i: (0, 0)),
        ],
        out_specs=[
            pl.BlockSpec((64, 64), lambda i: (0, 0)),
            pl.BlockSpec((8, 64), lambda i: (0, 0)),
        ],
        out_shape=[
            jax.ShapeDtypeStruct((64, 64), jnp.float32),
            jax.ShapeDtypeStruct((8, 64), jnp.float32),
        ],
    )(p_aug, w1b, m1r, sc1r, be1r)


def _mlp(p_aug, w1b, m1r, sc1r, be1r, w2b, cb2r, sc2r, be2r, w3b, b3r):
    return pl.pallas_call(
        _mlp_body,
        grid=(GRID,),
        in_specs=[
            pl.BlockSpec((BLK, 8), lambda i: (i, 0)),
            pl.BlockSpec((8, 64), lambda i: (0, 0)),
            pl.BlockSpec((8, 64), lambda i: (0, 0)),
            pl.BlockSpec((8, 64), lambda i: (0, 0)),
            pl.BlockSpec((8, 64), lambda i: (0, 0)),
            pl.BlockSpec((64, 128), lambda i: (0, 0)),
            pl.BlockSpec((8, 128), lambda i: (0, 0)),
            pl.BlockSpec((8, 128), lambda i: (0, 0)),
            pl.BlockSpec((8, 128), lambda i: (0, 0)),
            pl.BlockSpec((128, 384), lambda i: (0, 0)),
            pl.BlockSpec((8, 384), lambda i: (0, 0)),
        ],
        out_specs=pl.BlockSpec((BLK // PP, 384), lambda i: (i, 0)),
        out_shape=jax.ShapeDtypeStruct((ROWS // PP, 384), jnp.float32),
    )(p_aug, w1b, m1r, sc1r, be1r, w2b, cb2r, sc2r, be2r, w3b, b3r)


def kernel(x, W1, b1, g1, be1, W2, b2, g2, be2, W3, b3):
    eps = 1e-5
    f32 = jnp.float32
    xs = x[:, :, 0]
    ys = x[:, :, 1]
    zs = x[:, :, 2]
    cx, cy, cz, px, py, pz = _sc_call(xs, ys, zs)
    center_pos = jnp.stack([cx, cy, cz], axis=-1)

    p = jnp.stack([px, py, pz], axis=-1).reshape(ROWS, 3)
    p_aug = jnp.concatenate(
        [p, jnp.ones((ROWS, 1), f32), jnp.zeros((ROWS, 4), f32)],
        axis=-1).astype(jnp.bfloat16)

    rn = float(ROWS)
    w1a = jnp.zeros((8, 64), f32)
    w1a = w1a.at[0:3, :].set(W1.T)
    w1a = w1a.at[3, :].set(b1)
    w1b = w1a.astype(jnp.bfloat16)
    w1f = w1b.astype(f32)
    m0 = _moments(p_aug)
    m1 = (m0[:, 3] / rn) @ w1f
    e2 = jnp.sum(w1f * ((m0 / rn) @ w1f), axis=0)
    v1 = e2 - m1 * m1
    sc1 = g1 / jnp.sqrt(v1 + eps)

    m1r = _row8(m1, 64)
    sc1r = _row8(sc1, 64)
    be1r = _row8(be1, 64)

    w2b = W2.T.astype(jnp.bfloat16)
    w2f = w2b.astype(f32)
    gm, sm = _stats1(p_aug, w1b, m1r, sc1r, be1r)
    s1a = sm[0] / rn
    m2 = s1a @ w2f + b2
    e2b = (jnp.sum(w2f * ((gm / rn) @ w2f), axis=0)
           + 2.0 * b2 * (s1a @ w2f) + b2 * b2)
    v2 = e2b - m2 * m2
    sc2 = g2 / jnp.sqrt(v2 + eps)

    cb2r = _row8(b2 - m2, 128)
    sc2r = _row8(sc2, 128)
    be2r = _row8(be2, 128)
    w3b = W3.T.astype(jnp.bfloat16)
    b3r = _row8(b3, 384)

    out = _mlp(p_aug, w1b, m1r, sc1r, be1r, w2b, cb2r, sc2r, be2r, w3b, b3r)
    emb = out.reshape(B, NP, 384)
    return (emb, center_pos)


# speedup vs baseline: 1.0324x; 1.0324x over previous
import functools

import jax
import jax.numpy as jnp
from jax import lax
from jax.experimental import pallas as pl
from jax.experimental.pallas import tpu as pltpu
from jax.experimental.pallas import tpu_sc as plsc

B = 32
N = 2048
NP = 128
PP = 32
ROWS = B * NP * PP
BLK = 2048
GRID = ROWS // BLK
EPS = 1e-5
RN = float(ROWS)


def _trunc_bf16(v):
    bits = plsc.bitcast(v, jnp.int32)
    rounded = bits + 0x7FFF + ((bits >> 16) & 1)
    return plsc.bitcast(rounded & jnp.int32(-65536), jnp.float32)


def _sc_body(xs_h, ys_h, zs_h,
             c_h, p_h,
             xs_v, ys_v, zs_v, dist_v, xb_v, yb_v, zb_v, sp_v,
             cx_v, cy_v, cz_v, cbuf_v, pbuf_v):
    w = lax.axis_index("s") * 2 + lax.axis_index("c")
    pltpu.sync_copy(xs_h.at[w], xs_v)
    pltpu.sync_copy(ys_h.at[w], ys_v)
    pltpu.sync_copy(zs_h.at[w], zs_v)

    iota = lax.iota(jnp.int32, 16)
    lane0 = iota == 0
    inf16 = jnp.full((16,), 1e30, jnp.float32)
    ones16 = jnp.full((16,), 1.0, jnp.float32)
    zero16i = jnp.zeros((16,), jnp.int32)

    def init_chunk(c, carry):
        sl = pl.ds(c * 16, 16)
        dist_v[sl] = jnp.full((16,), 1e10, jnp.float32)
        xv = xs_v[sl]
        yv = ys_v[sl]
        zv = zs_v[sl]
        xb_v[sl] = _trunc_bf16(xv)
        yb_v[sl] = _trunc_bf16(yv)
        zb_v[sl] = _trunc_bf16(zv)
        sp_v[sl] = xv * xv + yv * yv + zv * zv
        return carry

    lax.fori_loop(0, N // 16, init_chunk, 0)

    def fps_step(t, far):
        fv = jnp.full((16,), far, jnp.int32)
        cxv = plsc.load_gather(xs_v, [fv])
        cyv = plsc.load_gather(ys_v, [fv])
        czv = plsc.load_gather(zs_v, [fv])
        tv = jnp.full((16,), t, jnp.int32)
        plsc.store_scatter(cx_v, [tv], cxv, mask=lane0)
        plsc.store_scatter(cy_v, [tv], cyv, mask=lane0)
        plsc.store_scatter(cz_v, [tv], czv, mask=lane0)
        t3 = jnp.full((16,), 3 * t, jnp.int32)
        plsc.store_scatter(cbuf_v, [t3], cxv, mask=lane0)
        plsc.store_scatter(cbuf_v, [t3 + 1], cyv, mask=lane0)
        plsc.store_scatter(cbuf_v, [t3 + 2], czv, mask=lane0)

        def chunk(c, carry):
            m, mi = carry
            sl = pl.ds(c * 16, 16)
            dx = xs_v[sl] - cxv
            dy = ys_v[sl] - cyv
            dz = zs_v[sl] - czv
            d = dx * dx + dy * dy + dz * dz
            dn = jnp.minimum(dist_v[sl], d)
            dist_v[sl] = dn
            upd = dn > m
            m = jnp.where(upd, dn, m)
            mi = jnp.where(upd, c * 16 + iota, mi)
            return (m, mi)

        m, mi = lax.fori_loop(0, N // 16, chunk,
                              (jnp.full((16,), -1.0, jnp.float32), zero16i))
        mmax = jnp.max(m)
        cand = jnp.where(m == mmax, mi, N)
        return jnp.min(cand)

    lax.fori_loop(0, NP, fps_step, jnp.int32(0))

    def knn_q(q, carry):
        qv = jnp.full((16,), q, jnp.int32)
        qx = plsc.load_gather(cx_v, [qv])
        qy = plsc.load_gather(cy_v, [qv])
        qz = plsc.load_gather(cz_v, [qv])
        qxb = _trunc_bf16(qx)
        qyb = _trunc_bf16(qy)
        qzb = _trunc_bf16(qz)
        sq = qx * qx + qy * qy + qz * qz

        def chunk(c, st):
            r0k, r0v, r1k, r1v = st
            sl = pl.ds(c * 16, 16)
            prod = xb_v[sl] * qxb + yb_v[sl] * qyb
            prod = prod + zb_v[sl] * qzb
            d = -2.0 * prod + sq
            d = d + sp_v[sl]
            ck, cv = plsc.sort_key_val(d, c * 16 + iota)
            rck = lax.rev(ck, (0,))
            rcv = lax.rev(cv, (0,))
            sel = r1k <= rck
            lk = jnp.where(sel, r1k, rck)
            lv = jnp.where(sel, r1v, rcv)
            lk, lv = plsc.sort_key_val(lk, lv)
            rlk = lax.rev(lk, (0,))
            rlv = lax.rev(lv, (0,))
            sel2 = r0k <= rlk
            lok = jnp.where(sel2, r0k, rlk)
            lov = jnp.where(sel2, r0v, rlv)
            hik = jnp.where(sel2, rlk, r0k)
            hiv = jnp.where(sel2, rlv, r0v)
            r0k, r0v = plsc.sort_key_val(lok, lov)
            r1k, r1v = plsc.sort_key_val(hik, hiv)
            return (r0k, r0v, r1k, r1v)

        r0k, r0v, r1k, r1v = lax.fori_loop(
            0, N // 16, chunk, (inf16, zero16i, inf16, zero16i))

        idx0 = q * (PP * 4) + iota * 4
        idx1 = idx0 + 64
        plsc.store_scatter(pbuf_v, [idx0], plsc.load_gather(xs_v, [r0v]) - qx)
        plsc.store_scatter(pbuf_v, [idx1], plsc.load_gather(xs_v, [r1v]) - qx)
        plsc.store_scatter(pbuf_v, [idx0 + 1], plsc.load_gather(ys_v, [r0v]) - qy)
        plsc.store_scatter(pbuf_v, [idx1 + 1], plsc.load_gather(ys_v, [r1v]) - qy)
        plsc.store_scatter(pbuf_v, [idx0 + 2], plsc.load_gather(zs_v, [r0v]) - qz)
        plsc.store_scatter(pbuf_v, [idx1 + 2], plsc.load_gather(zs_v, [r1v]) - qz)
        plsc.store_scatter(pbuf_v, [idx0 + 3], ones16)
        plsc.store_scatter(pbuf_v, [idx1 + 3], ones16)
        return carry

    lax.fori_loop(0, NP, knn_q, 0)

    pltpu.sync_copy(cbuf_v, c_h.at[w])
    pltpu.sync_copy(pbuf_v, p_h.at[w])


_sc_call = functools.partial(
    pl.kernel,
    mesh=plsc.VectorSubcoreMesh(core_axis_name="c", subcore_axis_name="s"),
    compiler_params=pltpu.CompilerParams(needs_layout_passes=False),
    out_type=[
        jax.ShapeDtypeStruct((B, NP * 3), jnp.float32),
        jax.ShapeDtypeStruct((B, NP * PP * 4), jnp.float32),
    ],
    scratch_types=[
        pltpu.VMEM((N,), jnp.float32),
        pltpu.VMEM((N,), jnp.float32),
        pltpu.VMEM((N,), jnp.float32),
        pltpu.VMEM((N,), jnp.float32),
        pltpu.VMEM((N,), jnp.float32),
        pltpu.VMEM((N,), jnp.float32),
        pltpu.VMEM((N,), jnp.float32),
        pltpu.VMEM((N,), jnp.float32),
        pltpu.VMEM((NP,), jnp.float32),
        pltpu.VMEM((NP,), jnp.float32),
        pltpu.VMEM((NP,), jnp.float32),
        pltpu.VMEM((NP * 3,), jnp.float32),
        pltpu.VMEM((NP * PP * 4,), jnp.float32),
    ],
)(_sc_body)


def _dotf(a, b):
    return lax.dot_general(a, b, (((1,), (0,)), ((), ())),
                           preferred_element_type=jnp.float32)


def _mom_body(p_ref, m_ref):
    @pl.when(pl.program_id(0) == 0)
    def _():
        m_ref[...] = jnp.zeros_like(m_ref)

    p = p_ref[...].astype(jnp.bfloat16)
    m_ref[...] += lax.dot_general(p, p, (((0,), (0,)), ((), ())),
                                  preferred_element_type=jnp.float32)


def _a1_of(p_ref, w1_ref, m0_ref, g1_ref, be1_ref):
    w1f = w1_ref[...].astype(jnp.float32)
    m0 = m0_ref[...] / RN
    m1 = jnp.sum(w1f * (m0[:, 3:4]), axis=0, keepdims=True)
    e2 = jnp.sum(w1f * _dotf(m0, w1f), axis=0, keepdims=True)
    v1 = e2 - m1 * m1
    sc1 = g1_ref[0:1, :] / jnp.sqrt(v1 + EPS)
    h1 = _dotf(p_ref[...].astype(jnp.bfloat16), w1_ref[...])
    return jnp.maximum((h1 - m1) * sc1 + be1_ref[0:1, :], 0.0)


def _stat1_body(p_ref, w1_ref, m0_ref, g1_ref, be1_ref, g_ref, s_ref):
    @pl.when(pl.program_id(0) == 0)
    def _():
        g_ref[...] = jnp.zeros_like(g_ref)
        s_ref[...] = jnp.zeros_like(s_ref)

    a1 = _a1_of(p_ref, w1_ref, m0_ref, g1_ref, be1_ref).astype(jnp.bfloat16)
    g_ref[...] += lax.dot_general(a1, a1, (((0,), (0,)), ((), ())),
                                  preferred_element_type=jnp.float32)
    s_ref[...] += jnp.broadcast_to(
        jnp.sum(a1.astype(jnp.float32), axis=0, keepdims=True), s_ref.shape)


def _mlp_body(p_ref, w1_ref, m0_ref, g1_ref, be1_ref,
              w2_ref, b2_ref, g2_ref, be2_ref, gm_ref, sm_ref,
              w3_ref, b3_ref, o_ref):
    a1 = _a1_of(p_ref, w1_ref, m0_ref, g1_ref, be1_ref).astype(jnp.bfloat16)

    w2f = w2_ref[...].astype(jnp.float32)
    b2 = b2_ref[0:1, :]
    t2 = _dotf(sm_ref[0:1, :] / RN, w2f)
    m2 = t2 + b2
    e2b = (jnp.sum(w2f * _dotf(gm_ref[...] / RN, w2f), axis=0, keepdims=True)
           + 2.0 * b2 * t2 + b2 * b2)
    v2 = e2b - m2 * m2
    sc2 = g2_ref[0:1, :] / jnp.sqrt(v2 + EPS)

    h2 = _dotf(a1, w2_ref[...])
    a2 = jnp.maximum((h2 + b2 - m2) * sc2 + be2_ref[0:1, :], 0.0)
    h3 = _dotf(a2.astype(jnp.bfloat16), w3_ref[...]) + b3_ref[0:1, :]
    o_ref[...] = jnp.max(h3.reshape(BLK // PP, PP, h3.shape[-1]), axis=1)


def _row8(v, n):
    return jnp.broadcast_to(v[None, :], (8, n)).astype(jnp.float32)


def _full(shape):
    return pl.BlockSpec(shape, lambda i: tuple(0 for _ in shape))


def _moments(p4):
    return pl.pallas_call(
        _mom_body,
        grid=(GRID,),
        in_specs=[pl.BlockSpec((BLK, 4), lambda i: (i, 0))],
        out_specs=_full((4, 4)),
        out_shape=jax.ShapeDtypeStruct((4, 4), jnp.float32),
    )(p4)


def _stats1(p4, w1b, m0, g1r, be1r):
    return pl.pallas_call(
        _stat1_body,
        grid=(GRID,),
        in_specs=[
            pl.BlockSpec((BLK, 4), lambda i: (i, 0)),
            _full((4, 64)),
            _full((4, 4)),
            _full((8, 64)),
            _full((8, 64)),
        ],
        out_specs=[_full((64, 64)), _full((8, 64))],
        out_shape=[
            jax.ShapeDtypeStruct((64, 64), jnp.float32),
            jax.ShapeDtypeStruct((8, 64), jnp.float32),
        ],
    )(p4, w1b, m0, g1r, be1r)


def _mlp(p4, w1b, m0, g1r, be1r, w2b, b2r, g2r, be2r, gm, sm, w3b, b3r):
    return pl.pallas_call(
        _mlp_body,
        grid=(GRID,),
        in_specs=[
            pl.BlockSpec((BLK, 4), lambda i: (i, 0)),
            _full((4, 64)),
            _full((4, 4)),
            _full((8, 64)),
            _full((8, 64)),
            _full((64, 128)),
            _full((8, 128)),
            _full((8, 128)),
            _full((8, 128)),
            _full((64, 64)),
            _full((8, 64)),
            _full((128, 384)),
            _full((8, 384)),
        ],
        out_specs=pl.BlockSpec((BLK // PP, 384), lambda i: (i, 0)),
        out_shape=jax.ShapeDtypeStruct((ROWS // PP, 384), jnp.float32),
    )(p4, w1b, m0, g1r, be1r, w2b, b2r, g2r, be2r, gm, sm, w3b, b3r)


def kernel(x, W1, b1, g1, be1, W2, b2, g2, be2, W3, b3):
    f32 = jnp.float32
    xs = x[:, :, 0]
    ys = x[:, :, 1]
    zs = x[:, :, 2]
    c_out, p_out = _sc_call(xs, ys, zs)
    center_pos = c_out.reshape(B, NP, 3)
    p4 = p_out.reshape(ROWS, 4)

    w1a = jnp.zeros((4, 64), f32)
    w1a = w1a.at[0:3, :].set(W1.T)
    w1a = w1a.at[3, :].set(b1)
    w1b = w1a.astype(jnp.bfloat16)
    w2b = W2.T.astype(jnp.bfloat16)
    w3b = W3.T.astype(jnp.bfloat16)
    g1r = _row8(g1, 64)
    be1r = _row8(be1, 64)
    b2r = _row8(b2, 128)
    g2r = _row8(g2, 128)
    be2r = _row8(be2, 128)
    b3r = _row8(b3, 384)

    m0 = _moments(p4)
    gm, sm = _stats1(p4, w1b, m0, g1r, be1r)
    out = _mlp(p4, w1b, m0, g1r, be1r, w2b, b2r, g2r, be2r, gm, sm, w3b, b3r)
    emb = out.reshape(B, NP, 384)
    return (emb, center_pos)


# speedup vs baseline: 1.2234x; 1.1850x over previous
import functools

import jax
import jax.numpy as jnp
from jax import lax
from jax.experimental import pallas as pl
from jax.experimental.pallas import tpu as pltpu
from jax.experimental.pallas import tpu_sc as plsc

B = 32
N = 2048
NP = 128
PP = 32
ROWS = B * NP * PP
BLK = 2048
GRID = ROWS // BLK
EPS = 1e-5
RN = float(ROWS)


def _trunc_bf16(v):
    bits = plsc.bitcast(v, jnp.int32)
    rounded = bits + 0x7FFF + ((bits >> 16) & 1)
    return plsc.bitcast(rounded & jnp.int32(-65536), jnp.float32)


def _sc_body(xs_h, ys_h, zs_h,
             c_h, p_h,
             xs_v, ys_v, zs_v, dist_v, xb_v, yb_v, zb_v, sp_v,
             cx_v, cy_v, cz_v, cbuf_v, pbuf_v, cand_v):
    w = lax.axis_index("s") * 2 + lax.axis_index("c")
    pltpu.sync_copy(xs_h.at[w], xs_v)
    pltpu.sync_copy(ys_h.at[w], ys_v)
    pltpu.sync_copy(zs_h.at[w], zs_v)

    iota = lax.iota(jnp.int32, 16)
    lane0 = iota == 0
    inf16 = jnp.full((16,), 1e30, jnp.float32)
    ones16 = jnp.full((16,), 1.0, jnp.float32)
    zero16i = jnp.zeros((16,), jnp.int32)

    def init_chunk(c, carry):
        sl = pl.ds(c * 16, 16)
        dist_v[sl] = jnp.full((16,), 1e10, jnp.float32)
        xv = xs_v[sl]
        yv = ys_v[sl]
        zv = zs_v[sl]
        xb_v[sl] = _trunc_bf16(xv)
        yb_v[sl] = _trunc_bf16(yv)
        zb_v[sl] = _trunc_bf16(zv)
        sp_v[sl] = xv * xv + yv * yv + zv * zv
        return carry

    lax.fori_loop(0, N // 16, init_chunk, 0)

    def fps_step(t, far):
        fv = jnp.full((16,), far, jnp.int32)
        cxv = plsc.load_gather(xs_v, [fv])
        cyv = plsc.load_gather(ys_v, [fv])
        czv = plsc.load_gather(zs_v, [fv])
        tv = jnp.full((16,), t, jnp.int32)
        plsc.store_scatter(cx_v, [tv], cxv, mask=lane0)
        plsc.store_scatter(cy_v, [tv], cyv, mask=lane0)
        plsc.store_scatter(cz_v, [tv], czv, mask=lane0)
        t3 = jnp.full((16,), 3 * t, jnp.int32)
        plsc.store_scatter(cbuf_v, [t3], cxv, mask=lane0)
        plsc.store_scatter(cbuf_v, [t3 + 1], cyv, mask=lane0)
        plsc.store_scatter(cbuf_v, [t3 + 2], czv, mask=lane0)

        def chunk(c, carry):
            m, mi = carry
            sl = pl.ds(c * 16, 16)
            dx = xs_v[sl] - cxv
            dy = ys_v[sl] - cyv
            dz = zs_v[sl] - czv
            d = dx * dx + dy * dy + dz * dz
            dn = jnp.minimum(dist_v[sl], d)
            dist_v[sl] = dn
            upd = dn > m
            m = jnp.where(upd, dn, m)
            mi = jnp.where(upd, c * 16 + iota, mi)
            return (m, mi)

        m, mi = lax.fori_loop(0, N // 16, chunk,
                              (jnp.full((16,), -1.0, jnp.float32), zero16i))
        mmax = jnp.max(m)
        cand = jnp.where(m == mmax, mi, N)
        return jnp.min(cand)

    lax.fori_loop(0, NP, fps_step, jnp.int32(0))

    def knn_q(q, carry):
        qv = jnp.full((16,), q, jnp.int32)
        qx = plsc.load_gather(cx_v, [qv])
        qy = plsc.load_gather(cy_v, [qv])
        qz = plsc.load_gather(cz_v, [qv])
        qxb = _trunc_bf16(qx)
        qyb = _trunc_bf16(qy)
        qzb = _trunc_bf16(qz)
        sq = qx * qx + qy * qy + qz * qz

        def pass_a(k, ms):
            new = []
            for r in range(8):
                sl = pl.ds((k * 8 + r) * 16, 16)
                prod = xb_v[sl] * qxb + yb_v[sl] * qyb
                prod = prod + zb_v[sl] * qzb
                d = -2.0 * prod + sq
                d = d + sp_v[sl]
                dist_v[sl] = d
                new.append(jnp.minimum(ms[r], d))
            return tuple(new)

        ms = lax.fori_loop(0, 16, pass_a, (inf16,) * 8)

        t0, t1 = inf16, inf16
        for r in range(8):
            ck = jnp.sort(ms[r])
            lk = jnp.sort(jnp.minimum(t1, lax.rev(ck, (0,))))
            rlk = lax.rev(lk, (0,))
            t0, t1 = (jnp.sort(jnp.minimum(t0, rlk)),
                      jnp.sort(jnp.maximum(t0, rlk)))
        tau = jnp.max(t1)

        cnts = []
        for r in range(8):
            mask = ms[r] <= tau
            plsc.store_compressed(cand_v.at[r], r * 16 + iota, mask=mask)
            cnts.append(jnp.max(plsc.all_reduce_population_count(mask)))

        def make_body(r):
            def body(j, st):
                r0k, r0v, r1k, r1v = st
                bv = plsc.load_gather(
                    cand_v, [jnp.full((16,), r, jnp.int32),
                             jnp.full((16,), j, jnp.int32)])
                pidx = iota * 128 + bv
                dv = plsc.load_gather(dist_v, [pidx])
                ck, cv = plsc.sort_key_val(dv, pidx)
                rck = lax.rev(ck, (0,))
                rcv = lax.rev(cv, (0,))
                sel = r1k <= rck
                lk = jnp.where(sel, r1k, rck)
                lv = jnp.where(sel, r1v, rcv)
                lk, lv = plsc.sort_key_val(lk, lv)
                rlk = lax.rev(lk, (0,))
                rlv = lax.rev(lv, (0,))
                sel2 = r0k <= rlk
                lok = jnp.where(sel2, r0k, rlk)
                lov = jnp.where(sel2, r0v, rlv)
                hik = jnp.where(sel2, rlk, r0k)
                hiv = jnp.where(sel2, rlv, r0v)
                r0k, r0v = plsc.sort_key_val(lok, lov)
                r1k, r1v = plsc.sort_key_val(hik, hiv)
                return (r0k, r0v, r1k, r1v)
            return body

        st = (inf16, zero16i, inf16, zero16i)
        for r in range(8):
            st = lax.fori_loop(0, cnts[r], make_body(r), st)
        r0k, r0v, r1k, r1v = st

        idx0 = q * (PP * 4) + iota * 4
        idx1 = idx0 + 64
        plsc.store_scatter(pbuf_v, [idx0], plsc.load_gather(xs_v, [r0v]) - qx)
        plsc.store_scatter(pbuf_v, [idx1], plsc.load_gather(xs_v, [r1v]) - qx)
        plsc.store_scatter(pbuf_v, [idx0 + 1], plsc.load_gather(ys_v, [r0v]) - qy)
        plsc.store_scatter(pbuf_v, [idx1 + 1], plsc.load_gather(ys_v, [r1v]) - qy)
        plsc.store_scatter(pbuf_v, [idx0 + 2], plsc.load_gather(zs_v, [r0v]) - qz)
        plsc.store_scatter(pbuf_v, [idx1 + 2], plsc.load_gather(zs_v, [r1v]) - qz)
        plsc.store_scatter(pbuf_v, [idx0 + 3], ones16)
        plsc.store_scatter(pbuf_v, [idx1 + 3], ones16)
        return carry

    lax.fori_loop(0, NP, knn_q, 0)

    pltpu.sync_copy(cbuf_v, c_h.at[w])
    pltpu.sync_copy(pbuf_v, p_h.at[w])


_sc_call = functools.partial(
    pl.kernel,
    mesh=plsc.VectorSubcoreMesh(core_axis_name="c", subcore_axis_name="s"),
    compiler_params=pltpu.CompilerParams(needs_layout_passes=False),
    out_type=[
        jax.ShapeDtypeStruct((B, NP * 3), jnp.float32),
        jax.ShapeDtypeStruct((B, NP * PP * 4), jnp.float32),
    ],
    scratch_types=[
        pltpu.VMEM((N,), jnp.float32),
        pltpu.VMEM((N,), jnp.float32),
        pltpu.VMEM((N,), jnp.float32),
        pltpu.VMEM((N,), jnp.float32),
        pltpu.VMEM((N,), jnp.float32),
        pltpu.VMEM((N,), jnp.float32),
        pltpu.VMEM((N,), jnp.float32),
        pltpu.VMEM((N,), jnp.float32),
        pltpu.VMEM((NP,), jnp.float32),
        pltpu.VMEM((NP,), jnp.float32),
        pltpu.VMEM((NP,), jnp.float32),
        pltpu.VMEM((NP * 3,), jnp.float32),
        pltpu.VMEM((NP * PP * 4,), jnp.float32),
        pltpu.VMEM((8, 16), jnp.int32),
    ],
)(_sc_body)


def _dotf(a, b):
    return lax.dot_general(a, b, (((1,), (0,)), ((), ())),
                           preferred_element_type=jnp.float32)


def _mom_body(p_ref, m_ref):
    @pl.when(pl.program_id(0) == 0)
    def _():
        m_ref[...] = jnp.zeros_like(m_ref)

    p = p_ref[...].astype(jnp.bfloat16)
    m_ref[...] += lax.dot_general(p, p, (((0,), (0,)), ((), ())),
                                  preferred_element_type=jnp.float32)


def _a1_of(p_ref, w1_ref, m0_ref, g1_ref, be1_ref):
    w1f = w1_ref[...].astype(jnp.float32)
    m0 = m0_ref[...] / RN
    m1 = jnp.sum(w1f * (m0[:, 3:4]), axis=0, keepdims=True)
    e2 = jnp.sum(w1f * _dotf(m0, w1f), axis=0, keepdims=True)
    v1 = e2 - m1 * m1
    sc1 = g1_ref[0:1, :] / jnp.sqrt(v1 + EPS)
    h1 = _dotf(p_ref[...].astype(jnp.bfloat16), w1_ref[...])
    return jnp.maximum((h1 - m1) * sc1 + be1_ref[0:1, :], 0.0)


def _stat1_body(p_ref, w1_ref, m0_ref, g1_ref, be1_ref, g_ref, s_ref):
    @pl.when(pl.program_id(0) == 0)
    def _():
        g_ref[...] = jnp.zeros_like(g_ref)
        s_ref[...] = jnp.zeros_like(s_ref)

    a1 = _a1_of(p_ref, w1_ref, m0_ref, g1_ref, be1_ref).astype(jnp.bfloat16)
    g_ref[...] += lax.dot_general(a1, a1, (((0,), (0,)), ((), ())),
                                  preferred_element_type=jnp.float32)
    s_ref[...] += jnp.broadcast_to(
        jnp.sum(a1.astype(jnp.float32), axis=0, keepdims=True), s_ref.shape)


def _mlp_body(p_ref, w1_ref, m0_ref, g1_ref, be1_ref,
              w2_ref, b2_ref, g2_ref, be2_ref, gm_ref, sm_ref,
              w3_ref, b3_ref, o_ref):
    a1 = _a1_of(p_ref, w1_ref, m0_ref, g1_ref, be1_ref).astype(jnp.bfloat16)

    w2f = w2_ref[...].astype(jnp.float32)
    b2 = b2_ref[0:1, :]
    t2 = _dotf(sm_ref[0:1, :] / RN, w2f)
    m2 = t2 + b2
    e2b = (jnp.sum(w2f * _dotf(gm_ref[...] / RN, w2f), axis=0, keepdims=True)
           + 2.0 * b2 * t2 + b2 * b2)
    v2 = e2b - m2 * m2
    sc2 = g2_ref[0:1, :] / jnp.sqrt(v2 + EPS)

    h2 = _dotf(a1, w2_ref[...])
    a2 = jnp.maximum((h2 + b2 - m2) * sc2 + be2_ref[0:1, :], 0.0)
    h3 = _dotf(a2.astype(jnp.bfloat16), w3_ref[...]) + b3_ref[0:1, :]
    o_ref[...] = jnp.max(h3.reshape(BLK // PP, PP, h3.shape[-1]), axis=1)


def _row8(v, n):
    return jnp.broadcast_to(v[None, :], (8, n)).astype(jnp.float32)


def _full(shape):
    return pl.BlockSpec(shape, lambda i: tuple(0 for _ in shape))


def _moments(p4):
    return pl.pallas_call(
        _mom_body,
        grid=(GRID,),
        in_specs=[pl.BlockSpec((BLK, 4), lambda i: (i, 0))],
        out_specs=_full((4, 4)),
        out_shape=jax.ShapeDtypeStruct((4, 4), jnp.float32),
    )(p4)


def _stats1(p4, w1b, m0, g1r, be1r):
    return pl.pallas_call(
        _stat1_body,
        grid=(GRID,),
        in_specs=[
            pl.BlockSpec((BLK, 4), lambda i: (i, 0)),
            _full((4, 64)),
            _full((4, 4)),
            _full((8, 64)),
            _full((8, 64)),
        ],
        out_specs=[_full((64, 64)), _full((8, 64))],
        out_shape=[
            jax.ShapeDtypeStruct((64, 64), jnp.float32),
            jax.ShapeDtypeStruct((8, 64), jnp.float32),
        ],
    )(p4, w1b, m0, g1r, be1r)


def _mlp(p4, w1b, m0, g1r, be1r, w2b, b2r, g2r, be2r, gm, sm, w3b, b3r):
    return pl.pallas_call(
        _mlp_body,
        grid=(GRID,),
        in_specs=[
            pl.BlockSpec((BLK, 4), lambda i: (i, 0)),
            _full((4, 64)),
            _full((4, 4)),
            _full((8, 64)),
            _full((8, 64)),
            _full((64, 128)),
            _full((8, 128)),
            _full((8, 128)),
            _full((8, 128)),
            _full((64, 64)),
            _full((8, 64)),
            _full((128, 384)),
            _full((8, 384)),
        ],
        out_specs=pl.BlockSpec((BLK // PP, 384), lambda i: (i, 0)),
        out_shape=jax.ShapeDtypeStruct((ROWS // PP, 384), jnp.float32),
    )(p4, w1b, m0, g1r, be1r, w2b, b2r, g2r, be2r, gm, sm, w3b, b3r)


def kernel(x, W1, b1, g1, be1, W2, b2, g2, be2, W3, b3):
    f32 = jnp.float32
    xs = x[:, :, 0]
    ys = x[:, :, 1]
    zs = x[:, :, 2]
    c_out, p_out = _sc_call(xs, ys, zs)
    center_pos = c_out.reshape(B, NP, 3)
    p4 = p_out.reshape(ROWS, 4)

    w1a = jnp.zeros((4, 64), f32)
    w1a = w1a.at[0:3, :].set(W1.T)
    w1a = w1a.at[3, :].set(b1)
    w1b = w1a.astype(jnp.bfloat16)
    w2b = W2.T.astype(jnp.bfloat16)
    w3b = W3.T.astype(jnp.bfloat16)
    g1r = _row8(g1, 64)
    be1r = _row8(be1, 64)
    b2r = _row8(b2, 128)
    g2r = _row8(g2, 128)
    be2r = _row8(be2, 128)
    b3r = _row8(b3, 384)

    m0 = _moments(p4)
    gm, sm = _stats1(p4, w1b, m0, g1r, be1r)
    out = _mlp(p4, w1b, m0, g1r, be1r, w2b, b2r, g2r, be2r, gm, sm, w3b, b3r)
    emb = out.reshape(B, NP, 384)
    return (emb, center_pos)
